# Initial kernel scaffold; baseline (speedup 1.0000x reference)
import jax
import jax.numpy as jnp
from jax.experimental import pallas as pl


def kernel(species_token, ability_token, item_token, move_tokens, species_onehot_w, all_abilities_w, abilities_onehot_w, all_items_w, items_onehot_w, all_moves_w, moves_onehot_w):
    raise NotImplementedError("write your pallas kernel here")


# speedup vs baseline: 2.7618x; 2.7618x over previous
import functools

import jax
import jax.numpy as jnp
from jax import lax
from jax.experimental import pallas as pl
from jax.experimental.pallas import tpu as pltpu
from jax.experimental.pallas import tpu_sc as plsc

_B = 1024
_T = 12
_NT = _B * _T
_NS = 2048
_DA = 256
_DI = 128
_DM = 1024

_NC = 2
_NSUB = 16
_NW = _NC * _NSUB
_PER_W = _NT // _NW
_CHUNK = 64
_NCHUNK = _PER_W // _CHUNK


def _sc_gather_body(tok_hbm, wa_hbm, wi_hbm, wm_hbm,
                    ga_hbm, gi_hbm, gm_hbm,
                    idx_v, a_v, i_v, m_v, sem_a, sem_i, sem_m):
    wid = lax.axis_index("s") * _NC + lax.axis_index("c")
    base = wid * _PER_W
    for c in range(_NCHUNK):
        off = base + c * _CHUNK
        pltpu.sync_copy(tok_hbm.at[pl.ds(off, _CHUNK)], idx_v)
        for k in range(_CHUNK // 16):
            v = idx_v[pl.ds(k * 16, 16)]
            idx_v[pl.ds(k * 16, 16)] = jnp.maximum(v - 1, 0)
        cp_a = pltpu.async_copy(wa_hbm.at[idx_v], a_v, sem_a)
        cp_i = pltpu.async_copy(wi_hbm.at[idx_v], i_v, sem_i)
        cp_m = pltpu.async_copy(wm_hbm.at[idx_v], m_v, sem_m)
        cp_a.wait()
        cp_i.wait()
        cp_m.wait()
        pltpu.sync_copy(a_v, ga_hbm.at[pl.ds(off, _CHUNK)])
        pltpu.sync_copy(i_v, gi_hbm.at[pl.ds(off, _CHUNK)])
        pltpu.sync_copy(m_v, gm_hbm.at[pl.ds(off, _CHUNK)])


@functools.cache
def _make_sc_gather():
    return pl.kernel(
        _sc_gather_body,
        out_type=(
            jax.ShapeDtypeStruct((_NT, _DA), jnp.float32),
            jax.ShapeDtypeStruct((_NT, _DI), jnp.float32),
            jax.ShapeDtypeStruct((_NT, _DM), jnp.float32),
        ),
        mesh=plsc.VectorSubcoreMesh(core_axis_name="c", subcore_axis_name="s",
                                    num_cores=_NC, num_subcores=_NSUB),
        scratch_types=(
            pltpu.VMEM((_CHUNK,), jnp.int32),
            pltpu.VMEM((_CHUNK, _DA), jnp.float32),
            pltpu.VMEM((_CHUNK, _DI), jnp.float32),
            pltpu.VMEM((_CHUNK, _DM), jnp.float32),
            pltpu.SemaphoreType.DMA,
            pltpu.SemaphoreType.DMA,
            pltpu.SemaphoreType.DMA,
        ),
    )


_BBLK = 32


def _tc_body(st_ref, at_ref, it_ref, mt_ref, ga_ref, gi_ref, gm_ref,
             wa_ref, wi_ref, wm_ref,
             sp_out, ab_out, im_out, mv_out):
    st = st_ref[...]
    known = st > 0
    knownf = known.astype(jnp.float32)

    iota_s = lax.broadcasted_iota(jnp.int32, (_BBLK, _T, _NS), 2)
    oh = jnp.where(iota_s == st - 1, knownf, 0.0)
    counts = oh.sum(axis=1, keepdims=True)
    kcnt = knownf.sum(axis=1, keepdims=True)
    t_unk = jnp.maximum(float(_NS) - kcnt, 1.0)
    u = (1.0 - counts) / t_unk
    sp_out[...] = jnp.where(known, oh, u)

    def prior(g_ref, w_ref):
        g = g_ref[...]
        team = (g * knownf).sum(axis=1, keepdims=True)
        colsum = w_ref[...].sum(axis=1, keepdims=True)
        unk = (colsum - team) / t_unk
        return jnp.where(known, g, unk)

    raw_a = prior(ga_ref, wa_ref)
    raw_a = raw_a / jnp.maximum(raw_a.sum(-1, keepdims=True), 1.0)
    at = at_ref[...]
    iota_a = lax.broadcasted_iota(jnp.int32, (_BBLK, _T, _DA), 2)
    a_oh = (iota_a == at - 1).astype(jnp.float32)
    ab_out[...] = jnp.where(at > 0, a_oh, raw_a)

    raw_i = prior(gi_ref, wi_ref)
    raw_i = raw_i / jnp.maximum(raw_i.sum(-1, keepdims=True), 1.0)
    it = it_ref[...]
    iota_i = lax.broadcasted_iota(jnp.int32, (_BBLK, _T, _DI), 2)
    i_oh = (iota_i == it - 1).astype(jnp.float32)
    im_out[...] = jnp.where(it > 0, i_oh, raw_i)

    all_unk = prior(gm_ref, wm_ref)
    mt = mt_ref[...]
    iota_m = lax.broadcasted_iota(jnp.int32, (_BBLK, _T, _DM), 2)
    m_known = jnp.zeros((_BBLK, _T, _DM), jnp.float32)
    for k in range(4):
        mk = mt[..., k:k + 1]
        m_known = m_known + jnp.where(
            (iota_m == mk - 1) & (mk > 0), 1.0, 0.0)
    m_unk = all_unk - m_known
    m_unk = m_unk / jnp.maximum(m_unk.sum(-1, keepdims=True), 1.0)
    num_missing = 4.0 - (m_known > 0).sum(-1, keepdims=True).astype(jnp.float32)
    move_mask = known & (mt.sum(-1, keepdims=True) != 0)
    mv_out[...] = jnp.where(move_mask,
                            m_known + num_missing * m_unk, 4.0 * m_unk)


def _tc_epilogue(st, at, it, mt, ga, gi, gm, wa, wi, wm):
    grid = (_B // _BBLK,)
    tok3 = lambda i: (i, 0, 0)
    full = lambda i: (0, 0, 0)
    return pl.pallas_call(
        _tc_body,
        grid=grid,
        in_specs=[
            pl.BlockSpec((_BBLK, _T, 1), tok3),
            pl.BlockSpec((_BBLK, _T, 1), tok3),
            pl.BlockSpec((_BBLK, _T, 1), tok3),
            pl.BlockSpec((_BBLK, _T, 4), tok3),
            pl.BlockSpec((_BBLK, _T, _DA), tok3),
            pl.BlockSpec((_BBLK, _T, _DI), tok3),
            pl.BlockSpec((_BBLK, _T, _DM), tok3),
            pl.BlockSpec((1, _NS, _DA), full),
            pl.BlockSpec((1, _NS, _DI), full),
            pl.BlockSpec((1, _NS, _DM), full),
        ],
        out_specs=[
            pl.BlockSpec((_BBLK, _T, _NS), tok3),
            pl.BlockSpec((_BBLK, _T, _DA), tok3),
            pl.BlockSpec((_BBLK, _T, _DI), tok3),
            pl.BlockSpec((_BBLK, _T, _DM), tok3),
        ],
        out_shape=[
            jax.ShapeDtypeStruct((_B, _T, _NS), jnp.float32),
            jax.ShapeDtypeStruct((_B, _T, _DA), jnp.float32),
            jax.ShapeDtypeStruct((_B, _T, _DI), jnp.float32),
            jax.ShapeDtypeStruct((_B, _T, _DM), jnp.float32),
        ],
    )(st, at, it, mt, ga, gi, gm, wa, wi, wm)


def kernel(species_token, ability_token, item_token, move_tokens,
           species_onehot_w, all_abilities_w, abilities_onehot_w,
           all_items_w, items_onehot_w, all_moves_w, moves_onehot_w):
    st_flat = species_token.reshape(-1).astype(jnp.int32)
    ga, gi, gm = _make_sc_gather()(st_flat, all_abilities_w, all_items_w,
                                   all_moves_w)
    return _tc_epilogue(
        species_token.reshape(_B, _T, 1), ability_token.reshape(_B, _T, 1),
        item_token.reshape(_B, _T, 1), move_tokens,
        ga.reshape(_B, _T, _DA), gi.reshape(_B, _T, _DI),
        gm.reshape(_B, _T, _DM),
        all_abilities_w.reshape(1, _NS, _DA),
        all_items_w.reshape(1, _NS, _DI),
        all_moves_w.reshape(1, _NS, _DM))


# speedup vs baseline: 7.3887x; 2.6753x over previous
import functools

import jax
import jax.numpy as jnp
from jax import lax
from jax.experimental import pallas as pl
from jax.experimental.pallas import tpu as pltpu
from jax.experimental.pallas import tpu_sc as plsc

_B = 1024
_T = 12
_NT = _B * _T
_NS = 2048
_DA = 256
_DI = 128
_DM = 1024

_NC = 2
_NSUB = 16
_NW = _NC * _NSUB
_PER_W = _NT // _NW
_CHUNK = 64
_NCHUNK = _PER_W // _CHUNK


def _sc_gather_body(tok_hbm, wa_hbm, wi_hbm, wm_hbm,
                    ga_hbm, gi_hbm, gm_hbm,
                    idx_v, a_v, i_v, m_v, sem_a, sem_i, sem_m):
    wid = lax.axis_index("s") * _NC + lax.axis_index("c")
    base = wid * _PER_W
    for c in range(_NCHUNK):
        off = base + c * _CHUNK
        pltpu.sync_copy(tok_hbm.at[pl.ds(off, _CHUNK)], idx_v)
        for k in range(_CHUNK // 16):
            v = idx_v[pl.ds(k * 16, 16)]
            idx_v[pl.ds(k * 16, 16)] = jnp.maximum(v - 1, 0)
        cp_a = pltpu.async_copy(wa_hbm.at[idx_v], a_v, sem_a)
        cp_i = pltpu.async_copy(wi_hbm.at[idx_v], i_v, sem_i)
        cp_m = pltpu.async_copy(wm_hbm.at[idx_v], m_v, sem_m)
        cp_a.wait()
        cp_i.wait()
        cp_m.wait()
        pltpu.sync_copy(a_v, ga_hbm.at[pl.ds(off, _CHUNK)])
        pltpu.sync_copy(i_v, gi_hbm.at[pl.ds(off, _CHUNK)])
        pltpu.sync_copy(m_v, gm_hbm.at[pl.ds(off, _CHUNK)])


@functools.cache
def _make_sc_gather():
    return pl.kernel(
        _sc_gather_body,
        out_type=(
            jax.ShapeDtypeStruct((_NT, _DA), jnp.float32),
            jax.ShapeDtypeStruct((_NT, _DI), jnp.float32),
            jax.ShapeDtypeStruct((_NT, _DM), jnp.float32),
        ),
        mesh=plsc.VectorSubcoreMesh(core_axis_name="c", subcore_axis_name="s",
                                    num_cores=_NC, num_subcores=_NSUB),
        scratch_types=(
            pltpu.VMEM((_CHUNK,), jnp.int32),
            pltpu.VMEM((_CHUNK, _DA), jnp.float32),
            pltpu.VMEM((_CHUNK, _DI), jnp.float32),
            pltpu.VMEM((_CHUNK, _DM), jnp.float32),
            pltpu.SemaphoreType.DMA,
            pltpu.SemaphoreType.DMA,
            pltpu.SemaphoreType.DMA,
        ),
    )


def _colsum_body(wa_ref, wi_ref, wm_ref, ca_out, ci_out, cm_out):
    ca_out[...] = wa_ref[...].sum(axis=1, keepdims=True)
    ci_out[...] = wi_ref[...].sum(axis=1, keepdims=True)
    cm_out[...] = wm_ref[...].sum(axis=1, keepdims=True)


def _colsums(wa, wi, wm):
    return pl.pallas_call(
        _colsum_body,
        out_shape=[
            jax.ShapeDtypeStruct((1, 1, _DA), jnp.float32),
            jax.ShapeDtypeStruct((1, 1, _DI), jnp.float32),
            jax.ShapeDtypeStruct((1, 1, _DM), jnp.float32),
        ],
    )(wa, wi, wm)


_BBLK = 32


def _tc_body(st_ref, at_ref, it_ref, mt_ref, ga_ref, gi_ref, gm_ref,
             ca_ref, ci_ref, cm_ref,
             sp_out, ab_out, im_out, mv_out):
    st = st_ref[...]
    known = st > 0
    knownf = known.astype(jnp.float32)

    iota_s = lax.broadcasted_iota(jnp.int32, (_T, _BBLK, _NS), 2)
    oh = (iota_s == st - 1).astype(jnp.float32)
    counts = oh.sum(axis=0, keepdims=True)
    kcnt = knownf.sum(axis=0, keepdims=True)
    t_unk = jnp.maximum(float(_NS) - kcnt, 1.0)
    u = (1.0 - counts) / t_unk
    sp_out[...] = jnp.where(known, oh, u)

    def prior(g_ref, c_ref):
        g = g_ref[...]
        team = (g * knownf).sum(axis=0, keepdims=True)
        unk = (c_ref[...] - team) / t_unk
        return jnp.where(known, g, unk)

    raw_a = prior(ga_ref, ca_ref)
    raw_a = raw_a / jnp.maximum(raw_a.sum(-1, keepdims=True), 1.0)
    at = at_ref[...]
    iota_a = lax.broadcasted_iota(jnp.int32, (_T, _BBLK, _DA), 2)
    a_oh = (iota_a == at - 1).astype(jnp.float32)
    ab_out[...] = jnp.where(at > 0, a_oh, raw_a)

    raw_i = prior(gi_ref, ci_ref)
    raw_i = raw_i / jnp.maximum(raw_i.sum(-1, keepdims=True), 1.0)
    it = it_ref[...]
    iota_i = lax.broadcasted_iota(jnp.int32, (_T, _BBLK, _DI), 2)
    i_oh = (iota_i == it - 1).astype(jnp.float32)
    im_out[...] = jnp.where(it > 0, i_oh, raw_i)

    all_unk = prior(gm_ref, cm_ref)
    mt = mt_ref[...]
    iota_m = lax.broadcasted_iota(jnp.int32, (_T, _BBLK, _DM), 2)
    m_known = jnp.zeros((_T, _BBLK, _DM), jnp.float32)
    for k in range(4):
        mk = mt[..., k:k + 1]
        m_known = m_known + (iota_m == mk - 1).astype(jnp.float32)
    m_unk = all_unk - m_known
    m_unk = m_unk / jnp.maximum(m_unk.sum(-1, keepdims=True), 1.0)
    num_missing = 4.0 - (m_known > 0).sum(-1, keepdims=True).astype(jnp.float32)
    move_mask = known & (mt.sum(-1, keepdims=True) != 0)
    mv_out[...] = jnp.where(move_mask,
                            m_known + num_missing * m_unk, 4.0 * m_unk)


def _tc_epilogue(st, at, it, mt, ga, gi, gm, ca, ci, cm):
    grid = (_B // _BBLK,)
    tok3 = lambda i: (0, i, 0)
    full = lambda i: (0, 0, 0)
    return pl.pallas_call(
        _tc_body,
        grid=grid,
        in_specs=[
            pl.BlockSpec((_T, _BBLK, 1), tok3),
            pl.BlockSpec((_T, _BBLK, 1), tok3),
            pl.BlockSpec((_T, _BBLK, 1), tok3),
            pl.BlockSpec((_T, _BBLK, 4), tok3),
            pl.BlockSpec((_T, _BBLK, _DA), tok3),
            pl.BlockSpec((_T, _BBLK, _DI), tok3),
            pl.BlockSpec((_T, _BBLK, _DM), tok3),
            pl.BlockSpec((1, 1, _DA), full),
            pl.BlockSpec((1, 1, _DI), full),
            pl.BlockSpec((1, 1, _DM), full),
        ],
        out_specs=[
            pl.BlockSpec((_T, _BBLK, _NS), tok3),
            pl.BlockSpec((_T, _BBLK, _DA), tok3),
            pl.BlockSpec((_T, _BBLK, _DI), tok3),
            pl.BlockSpec((_T, _BBLK, _DM), tok3),
        ],
        out_shape=[
            jax.ShapeDtypeStruct((_T, _B, _NS), jnp.float32),
            jax.ShapeDtypeStruct((_T, _B, _DA), jnp.float32),
            jax.ShapeDtypeStruct((_T, _B, _DI), jnp.float32),
            jax.ShapeDtypeStruct((_T, _B, _DM), jnp.float32),
        ],
    )(st, at, it, mt, ga, gi, gm, ca, ci, cm)


def kernel(species_token, ability_token, item_token, move_tokens,
           species_onehot_w, all_abilities_w, abilities_onehot_w,
           all_items_w, items_onehot_w, all_moves_w, moves_onehot_w):
    st_t = species_token.T.astype(jnp.int32)
    at_t = ability_token.T.astype(jnp.int32)
    it_t = item_token.T.astype(jnp.int32)
    mt_t = jnp.transpose(move_tokens, (1, 0, 2)).astype(jnp.int32)
    ga, gi, gm = _make_sc_gather()(st_t.reshape(-1), all_abilities_w,
                                   all_items_w, all_moves_w)
    ca, ci, cm = _colsums(all_abilities_w.reshape(1, _NS, _DA),
                          all_items_w.reshape(1, _NS, _DI),
                          all_moves_w.reshape(1, _NS, _DM))
    sp, ab, im, mv = _tc_epilogue(
        st_t.reshape(_T, _B, 1), at_t.reshape(_T, _B, 1),
        it_t.reshape(_T, _B, 1), mt_t,
        ga.reshape(_T, _B, _DA), gi.reshape(_T, _B, _DI),
        gm.reshape(_T, _B, _DM), ca, ci, cm)
    tr = lambda x: jnp.transpose(x, (1, 0, 2))
    return tr(sp), tr(ab), tr(im), tr(mv)


# speedup vs baseline: 7.5590x; 1.0231x over previous
import functools

import jax
import jax.numpy as jnp
from jax import lax
from jax.experimental import pallas as pl
from jax.experimental.pallas import tpu as pltpu
from jax.experimental.pallas import tpu_sc as plsc

_B = 1024
_T = 12
_NT = _B * _T
_NS = 2048
_DA = 256
_DI = 128
_DM = 1024

_NC = 2
_NSUB = 16
_NW = _NC * _NSUB
_PER_W = _NT // _NW
_CHUNK = 32
_NCHUNK = _PER_W // _CHUNK


def _sc_gather_body(tok_hbm, wa_hbm, wi_hbm, wm_hbm,
                    ga_hbm, gi_hbm, gm_hbm,
                    idx_v, a0, i0, m0, a1, i1, m1,
                    sa0, si0, sm0, sa1, si1, sm1):
    wid = lax.axis_index("s") * _NC + lax.axis_index("c")
    base = wid * _PER_W
    pltpu.sync_copy(tok_hbm.at[pl.ds(base, _PER_W)], idx_v)
    for k in range(_PER_W // 16):
        v = idx_v[pl.ds(k * 16, 16)]
        idx_v[pl.ds(k * 16, 16)] = jnp.maximum(v - 1, 0)
    bufs = ((a0, i0, m0), (a1, i1, m1))
    sems = ((sa0, si0, sm0), (sa1, si1, sm1))

    def start(c):
        a_v, i_v, m_v = bufs[c % 2]
        sa, si, sm = sems[c % 2]
        ix = idx_v.at[pl.ds(c * _CHUNK, _CHUNK)]
        return (pltpu.async_copy(wa_hbm.at[ix], a_v, sa),
                pltpu.async_copy(wi_hbm.at[ix], i_v, si),
                pltpu.async_copy(wm_hbm.at[ix], m_v, sm))

    inflight = start(0)
    for c in range(_NCHUNK):
        for cp in inflight:
            cp.wait()
        inflight = start(c + 1) if c + 1 < _NCHUNK else ()
        a_v, i_v, m_v = bufs[c % 2]
        off = base + c * _CHUNK
        pltpu.sync_copy(a_v, ga_hbm.at[pl.ds(off, _CHUNK)])
        pltpu.sync_copy(i_v, gi_hbm.at[pl.ds(off, _CHUNK)])
        pltpu.sync_copy(m_v, gm_hbm.at[pl.ds(off, _CHUNK)])


@functools.cache
def _make_sc_gather():
    return pl.kernel(
        _sc_gather_body,
        out_type=(
            jax.ShapeDtypeStruct((_NT, _DA), jnp.float32),
            jax.ShapeDtypeStruct((_NT, _DI), jnp.float32),
            jax.ShapeDtypeStruct((_NT, _DM), jnp.float32),
        ),
        mesh=plsc.VectorSubcoreMesh(core_axis_name="c", subcore_axis_name="s",
                                    num_cores=_NC, num_subcores=_NSUB),
        scratch_types=(
            pltpu.VMEM((_PER_W,), jnp.int32),
            pltpu.VMEM((_CHUNK, _DA), jnp.float32),
            pltpu.VMEM((_CHUNK, _DI), jnp.float32),
            pltpu.VMEM((_CHUNK, _DM), jnp.float32),
            pltpu.VMEM((_CHUNK, _DA), jnp.float32),
            pltpu.VMEM((_CHUNK, _DI), jnp.float32),
            pltpu.VMEM((_CHUNK, _DM), jnp.float32),
            pltpu.SemaphoreType.DMA,
            pltpu.SemaphoreType.DMA,
            pltpu.SemaphoreType.DMA,
            pltpu.SemaphoreType.DMA,
            pltpu.SemaphoreType.DMA,
            pltpu.SemaphoreType.DMA,
        ),
    )


def _colsum_body(wa_ref, wi_ref, wm_ref, ca_out, ci_out, cm_out):
    ca_out[...] = wa_ref[...].sum(axis=1, keepdims=True)
    ci_out[...] = wi_ref[...].sum(axis=1, keepdims=True)
    cm_out[...] = wm_ref[...].sum(axis=1, keepdims=True)


def _colsums(wa, wi, wm):
    return pl.pallas_call(
        _colsum_body,
        out_shape=[
            jax.ShapeDtypeStruct((1, 1, _DA), jnp.float32),
            jax.ShapeDtypeStruct((1, 1, _DI), jnp.float32),
            jax.ShapeDtypeStruct((1, 1, _DM), jnp.float32),
        ],
    )(wa, wi, wm)


_BBLK = 32


def _tc_body(st_ref, at_ref, it_ref, mt_ref, ga_ref, gi_ref, gm_ref,
             ca_ref, ci_ref, cm_ref,
             sp_out, ab_out, im_out, mv_out):
    st = st_ref[...]
    known = st > 0
    knownf = known.astype(jnp.float32)

    iota_s = lax.broadcasted_iota(jnp.int32, (_T, _BBLK, _NS), 2)
    oh = (iota_s == st - 1).astype(jnp.float32)
    counts = oh.sum(axis=0, keepdims=True)
    kcnt = knownf.sum(axis=0, keepdims=True)
    t_unk = jnp.maximum(float(_NS) - kcnt, 1.0)
    u = (1.0 - counts) / t_unk
    sp_out[...] = jnp.where(known, oh, u)

    def prior(g_ref, c_ref):
        g = g_ref[...]
        team = (g * knownf).sum(axis=0, keepdims=True)
        unk = (c_ref[...] - team) / t_unk
        return jnp.where(known, g, unk)

    raw_a = prior(ga_ref, ca_ref)
    raw_a = raw_a / jnp.maximum(raw_a.sum(-1, keepdims=True), 1.0)
    at = at_ref[...]
    iota_a = lax.broadcasted_iota(jnp.int32, (_T, _BBLK, _DA), 2)
    a_oh = (iota_a == at - 1).astype(jnp.float32)
    ab_out[...] = jnp.where(at > 0, a_oh, raw_a)

    raw_i = prior(gi_ref, ci_ref)
    raw_i = raw_i / jnp.maximum(raw_i.sum(-1, keepdims=True), 1.0)
    it = it_ref[...]
    iota_i = lax.broadcasted_iota(jnp.int32, (_T, _BBLK, _DI), 2)
    i_oh = (iota_i == it - 1).astype(jnp.float32)
    im_out[...] = jnp.where(it > 0, i_oh, raw_i)

    all_unk = prior(gm_ref, cm_ref)
    mt = mt_ref[...]
    iota_m = lax.broadcasted_iota(jnp.int32, (_T, _BBLK, _DM), 2)
    m_known = jnp.zeros((_T, _BBLK, _DM), jnp.float32)
    for k in range(4):
        mk = mt[..., k:k + 1]
        m_known = m_known + (iota_m == mk - 1).astype(jnp.float32)
    m_unk = all_unk - m_known
    m_unk = m_unk / jnp.maximum(m_unk.sum(-1, keepdims=True), 1.0)
    num_missing = 4.0 - (m_known > 0).sum(-1, keepdims=True).astype(jnp.float32)
    move_mask = known & (mt.sum(-1, keepdims=True) != 0)
    mv_out[...] = jnp.where(move_mask,
                            m_known + num_missing * m_unk, 4.0 * m_unk)


def _tc_epilogue(st, at, it, mt, ga, gi, gm, ca, ci, cm):
    grid = (_B // _BBLK,)
    tok3 = lambda i: (0, i, 0)
    full = lambda i: (0, 0, 0)
    return pl.pallas_call(
        _tc_body,
        grid=grid,
        in_specs=[
            pl.BlockSpec((_T, _BBLK, 1), tok3),
            pl.BlockSpec((_T, _BBLK, 1), tok3),
            pl.BlockSpec((_T, _BBLK, 1), tok3),
            pl.BlockSpec((_T, _BBLK, 4), tok3),
            pl.BlockSpec((_T, _BBLK, _DA), tok3),
            pl.BlockSpec((_T, _BBLK, _DI), tok3),
            pl.BlockSpec((_T, _BBLK, _DM), tok3),
            pl.BlockSpec((1, 1, _DA), full),
            pl.BlockSpec((1, 1, _DI), full),
            pl.BlockSpec((1, 1, _DM), full),
        ],
        out_specs=[
            pl.BlockSpec((_T, _BBLK, _NS), tok3),
            pl.BlockSpec((_T, _BBLK, _DA), tok3),
            pl.BlockSpec((_T, _BBLK, _DI), tok3),
            pl.BlockSpec((_T, _BBLK, _DM), tok3),
        ],
        out_shape=[
            jax.ShapeDtypeStruct((_T, _B, _NS), jnp.float32),
            jax.ShapeDtypeStruct((_T, _B, _DA), jnp.float32),
            jax.ShapeDtypeStruct((_T, _B, _DI), jnp.float32),
            jax.ShapeDtypeStruct((_T, _B, _DM), jnp.float32),
        ],
    )(st, at, it, mt, ga, gi, gm, ca, ci, cm)


def kernel(species_token, ability_token, item_token, move_tokens,
           species_onehot_w, all_abilities_w, abilities_onehot_w,
           all_items_w, items_onehot_w, all_moves_w, moves_onehot_w):
    st_t = species_token.T.astype(jnp.int32)
    at_t = ability_token.T.astype(jnp.int32)
    it_t = item_token.T.astype(jnp.int32)
    mt_t = jnp.transpose(move_tokens, (1, 0, 2)).astype(jnp.int32)
    ga, gi, gm = _make_sc_gather()(st_t.reshape(-1), all_abilities_w,
                                   all_items_w, all_moves_w)
    ca, ci, cm = _colsums(all_abilities_w.reshape(1, _NS, _DA),
                          all_items_w.reshape(1, _NS, _DI),
                          all_moves_w.reshape(1, _NS, _DM))
    sp, ab, im, mv = _tc_epilogue(
        st_t.reshape(_T, _B, 1), at_t.reshape(_T, _B, 1),
        it_t.reshape(_T, _B, 1), mt_t,
        ga.reshape(_T, _B, _DA), gi.reshape(_T, _B, _DI),
        gm.reshape(_T, _B, _DM), ca, ci, cm)
    tr = lambda x: jnp.transpose(x, (1, 0, 2))
    return tr(sp), tr(ab), tr(im), tr(mv)
